# baseline (speedup 1.0000x reference)
import jax
import jax.numpy as jnp
from jax.experimental import pallas as pl

_N_EXPERTS = 4
_HIDDEN = 16
_OMEGA = 30.0
_BLOCK = 8192


def _bf16_round(v):
    return v.astype(jnp.bfloat16).astype(jnp.float32)


def _fused_kernel(x_ref, w1t_ref, b1t_ref, w2t_ref, b2_ref, lo_ref, hi_ref,
                  sc_ref, out_ref):
    xt = jnp.transpose(x_ref[...])
    xd = [xt[0:1, :], xt[1:2, :], xt[2:3, :]]

    sels = [None] * _N_EXPERTS
    claimed = None
    for e in range(_N_EXPERTS - 1, -1, -1):
        m = None
        for d in range(3):
            md = ((xd[d] >= lo_ref[e:e + 1, d:d + 1])
                  & (xd[d] < hi_ref[e:e + 1, d:d + 1]))
            m = md if m is None else (m & md)
        mf = m.astype(jnp.float32)
        if claimed is None:
            sels[e] = mf
            claimed = mf
        else:
            sels[e] = mf * (1.0 - claimed)
            claimed = jnp.maximum(claimed, mf)

    z = None
    for d in range(3):
        xn = None
        wsel = None
        for e in range(_N_EXPERTS):
            lo = lo_ref[e:e + 1, d:d + 1]
            hi = hi_ref[e:e + 1, d:d + 1]
            xne = (2.0 * (xd[d] - lo) / (hi - lo) - 1.0) * sc_ref[e:e + 1,
                                                                  d:d + 1]
            xn = sels[e] * xne if xn is None else (xn + sels[e] * xne)
            w1col = _bf16_round(w1t_ref[e * _HIDDEN:(e + 1) * _HIDDEN,
                                        d:d + 1])
            wsel = (sels[e] * w1col if wsel is None
                    else (wsel + sels[e] * w1col))
        term = _bf16_round(xn) * wsel
        z = term if z is None else (z + term)
    b1sel = None
    w2sel = None
    b2sel = None
    for e in range(_N_EXPERTS):
        b1col = b1t_ref[e * _HIDDEN:(e + 1) * _HIDDEN, :]
        w2col = _bf16_round(w2t_ref[e * _HIDDEN:(e + 1) * _HIDDEN, :])
        b2e = b2_ref[e:e + 1, :]
        if b1sel is None:
            b1sel = sels[e] * b1col
            w2sel = sels[e] * w2col
            b2sel = sels[e] * b2e
        else:
            b1sel = b1sel + sels[e] * b1col
            w2sel = w2sel + sels[e] * w2col
            b2sel = b2sel + sels[e] * b2e
    z = z + b1sel
    h = jnp.sin(_OMEGA * z)
    p = _bf16_round(h) * w2sel
    v = jnp.sum(p, axis=0, keepdims=True) + b2sel
    out_ref[...] = (claimed * v)[0, :]


@jax.jit
def kernel(x, children_meta, input_scale, W1, b1, W2, b2):
    if x.ndim == 1:
        x = x[None, :]
    n = x.shape[0]

    lo = children_meta[:, :, 0]
    hi = children_meta[:, :, 1]
    w1t = jnp.transpose(W1, (0, 2, 1)).reshape(_N_EXPERTS * _HIDDEN, 3)
    b1t = b1.reshape(_N_EXPERTS * _HIDDEN, 1)
    w2t = W2.reshape(_N_EXPERTS * _HIDDEN, 1)
    b2c = b2.reshape(_N_EXPERTS, 1)

    grid = (n // _BLOCK,)
    out = pl.pallas_call(
        _fused_kernel,
        grid=grid,
        in_specs=[
            pl.BlockSpec((_BLOCK, 3), lambda i: (i, 0)),
            pl.BlockSpec((_N_EXPERTS * _HIDDEN, 3), lambda i: (0, 0)),
            pl.BlockSpec((_N_EXPERTS * _HIDDEN, 1), lambda i: (0, 0)),
            pl.BlockSpec((_N_EXPERTS * _HIDDEN, 1), lambda i: (0, 0)),
            pl.BlockSpec((_N_EXPERTS, 1), lambda i: (0, 0)),
            pl.BlockSpec((_N_EXPERTS, 3), lambda i: (0, 0)),
            pl.BlockSpec((_N_EXPERTS, 3), lambda i: (0, 0)),
            pl.BlockSpec((_N_EXPERTS, 3), lambda i: (0, 0)),
        ],
        out_specs=pl.BlockSpec((_BLOCK,), lambda i: (i,)),
        out_shape=jax.ShapeDtypeStruct((n,), jnp.float32),
    )(x, w1t, b1t, w2t, b2c, lo, hi, input_scale)
    return out


# speedup vs baseline: 1.0605x; 1.0605x over previous
import jax
import jax.numpy as jnp
from jax.experimental import pallas as pl

_N_EXPERTS = 4
_HIDDEN = 16
_OMEGA = 30.0
_BLOCK = 4096


def _bf16_round(v):
    return v.astype(jnp.bfloat16).astype(jnp.float32)


def _fused_kernel(x_ref, w1t_ref, b1t_ref, w2t_ref, b2_ref, lo_ref, hi_ref,
                  sc_ref, out_ref):
    xt = jnp.transpose(x_ref[...])
    xd = [xt[0:1, :], xt[1:2, :], xt[2:3, :]]

    sels = [None] * _N_EXPERTS
    claimed = None
    for e in range(_N_EXPERTS - 1, -1, -1):
        m = None
        for d in range(3):
            md = ((xd[d] >= lo_ref[e:e + 1, d:d + 1])
                  & (xd[d] < hi_ref[e:e + 1, d:d + 1]))
            m = md if m is None else (m & md)
        mf = m.astype(jnp.float32)
        if claimed is None:
            sels[e] = mf
            claimed = mf
        else:
            sels[e] = mf * (1.0 - claimed)
            claimed = jnp.maximum(claimed, mf)

    z = None
    for d in range(3):
        xn = None
        wsel = None
        for e in range(_N_EXPERTS):
            lo = lo_ref[e:e + 1, d:d + 1]
            hi = hi_ref[e:e + 1, d:d + 1]
            xne = (2.0 * (xd[d] - lo) / (hi - lo) - 1.0) * sc_ref[e:e + 1,
                                                                  d:d + 1]
            xn = sels[e] * xne if xn is None else (xn + sels[e] * xne)
            w1col = _bf16_round(w1t_ref[e * _HIDDEN:(e + 1) * _HIDDEN,
                                        d:d + 1])
            wsel = (sels[e] * w1col if wsel is None
                    else (wsel + sels[e] * w1col))
        term = _bf16_round(xn) * wsel
        z = term if z is None else (z + term)
    b1sel = None
    w2sel = None
    b2sel = None
    for e in range(_N_EXPERTS):
        b1col = b1t_ref[e * _HIDDEN:(e + 1) * _HIDDEN, :]
        w2col = _bf16_round(w2t_ref[e * _HIDDEN:(e + 1) * _HIDDEN, :])
        b2e = b2_ref[e:e + 1, :]
        if b1sel is None:
            b1sel = sels[e] * b1col
            w2sel = sels[e] * w2col
            b2sel = sels[e] * b2e
        else:
            b1sel = b1sel + sels[e] * b1col
            w2sel = w2sel + sels[e] * w2col
            b2sel = b2sel + sels[e] * b2e
    z = z + b1sel
    h = jnp.sin(_OMEGA * z)
    p = _bf16_round(h) * w2sel
    v = jnp.sum(p, axis=0, keepdims=True) + b2sel
    out_ref[...] = (claimed * v)[0, :]


@jax.jit
def kernel(x, children_meta, input_scale, W1, b1, W2, b2):
    if x.ndim == 1:
        x = x[None, :]
    n = x.shape[0]

    lo = children_meta[:, :, 0]
    hi = children_meta[:, :, 1]
    w1t = jnp.transpose(W1, (0, 2, 1)).reshape(_N_EXPERTS * _HIDDEN, 3)
    b1t = b1.reshape(_N_EXPERTS * _HIDDEN, 1)
    w2t = W2.reshape(_N_EXPERTS * _HIDDEN, 1)
    b2c = b2.reshape(_N_EXPERTS, 1)

    grid = (n // _BLOCK,)
    out = pl.pallas_call(
        _fused_kernel,
        grid=grid,
        in_specs=[
            pl.BlockSpec((_BLOCK, 3), lambda i: (i, 0)),
            pl.BlockSpec((_N_EXPERTS * _HIDDEN, 3), lambda i: (0, 0)),
            pl.BlockSpec((_N_EXPERTS * _HIDDEN, 1), lambda i: (0, 0)),
            pl.BlockSpec((_N_EXPERTS * _HIDDEN, 1), lambda i: (0, 0)),
            pl.BlockSpec((_N_EXPERTS, 1), lambda i: (0, 0)),
            pl.BlockSpec((_N_EXPERTS, 3), lambda i: (0, 0)),
            pl.BlockSpec((_N_EXPERTS, 3), lambda i: (0, 0)),
            pl.BlockSpec((_N_EXPERTS, 3), lambda i: (0, 0)),
        ],
        out_specs=pl.BlockSpec((_BLOCK,), lambda i: (i,)),
        out_shape=jax.ShapeDtypeStruct((n,), jnp.float32),
    )(x, w1t, b1t, w2t, b2c, lo, hi, input_scale)
    return out
